# Initial kernel scaffold; baseline (speedup 1.0000x reference)
import jax
import jax.numpy as jnp
from jax.experimental import pallas as pl


def kernel(node_emb, rel_table, norm_table, edge_id, src, dst):
    raise NotImplementedError("write your pallas kernel here")


# speedup vs baseline: 11.6718x; 11.6718x over previous
import functools

import jax
import jax.numpy as jnp
from jax import lax
from jax.experimental import pallas as pl
from jax.experimental.pallas import tpu as pltpu
from jax.experimental.pallas import tpu_sc as plsc

_DIM = 128
_GAMMA = 12.0
_NW = 32
_C = 80


def _prep_body(rel_ref, norm_ref, out_ref):
    x = norm_ref[...]
    ss = jnp.sum(x * x, axis=-1, keepdims=True)
    inv = lax.rsqrt(jnp.maximum(ss, 1e-24))
    out_ref[:, :_DIM] = rel_ref[...]
    out_ref[:, _DIM:] = x * inv


def _prep(rel_table, norm_table):
    n_rel = rel_table.shape[0]
    return pl.pallas_call(
        _prep_body,
        out_shape=jax.ShapeDtypeStruct((n_rel, 2 * _DIM), jnp.float32),
    )(rel_table, norm_table)


@functools.cache
def _make_sc_kernel(n_edges):
    epw = n_edges // _NW
    n_chunks = epw // _C

    mesh = plsc.VectorSubcoreMesh(core_axis_name="c", subcore_axis_name="s")

    @functools.partial(
        pl.kernel,
        out_type=jax.ShapeDtypeStruct((_NW, epw), jnp.float32),
        mesh=mesh,
        compiler_params=pltpu.CompilerParams(needs_layout_passes=False),
        scratch_types=[
            pltpu.VMEM((4, 3, _C), jnp.int32),
            pltpu.VMEM((2, _C, _DIM), jnp.float32),
            pltpu.VMEM((2, _C, _DIM), jnp.float32),
            pltpu.VMEM((2, _C, 2 * _DIM), jnp.float32),
            pltpu.VMEM((epw,), jnp.float32),
            pltpu.SemaphoreType.DMA((4,)),
            pltpu.SemaphoreType.DMA((2,)),
        ],
    )
    def sc_kernel(node_hbm, relnorm_hbm, idx_hbm, out_hbm,
                  idx4, h2, t2, nr2, o_flat, isems, gsems):
        wid = lax.axis_index("s") * 2 + lax.axis_index("c")

        def issue_idx(ci):
            islot = ci % 4
            pltpu.async_copy(idx_hbm.at[wid, ci], idx4.at[islot],
                             isems.at[islot])

        def wait_idx(ci):
            islot = ci % 4
            pltpu.make_async_copy(idx_hbm.at[wid, ci], idx4.at[islot],
                                  isems.at[islot]).wait()

        def issue(ci, slot):
            islot = ci % 4
            pltpu.async_copy(node_hbm.at[idx4.at[islot, 0]], h2.at[slot],
                             gsems.at[slot])
            pltpu.async_copy(node_hbm.at[idx4.at[islot, 1]], t2.at[slot],
                             gsems.at[slot])
            pltpu.async_copy(relnorm_hbm.at[idx4.at[islot, 2]], nr2.at[slot],
                             gsems.at[slot])

        def wait(ci, slot):
            islot = ci % 4
            pltpu.make_async_copy(node_hbm.at[idx4.at[islot, 0]], h2.at[slot],
                                  gsems.at[slot]).wait()
            pltpu.make_async_copy(node_hbm.at[idx4.at[islot, 1]], t2.at[slot],
                                  gsems.at[slot]).wait()
            pltpu.make_async_copy(relnorm_hbm.at[idx4.at[islot, 2]],
                                  nr2.at[slot], gsems.at[slot]).wait()

        def process(ci, slot):
            h_v, t_v, nr_v = h2.at[slot], t2.at[slot], nr2.at[slot]
            lane = lax.iota(jnp.int32, 16)

            def group(g, carry):
                def edge(k, acc):
                    e = g * 16 + k
                    dj = []
                    nj = []
                    accd = jnp.zeros((16,), jnp.float32)
                    for j in range(_DIM // 16):
                        h = h_v[e, pl.ds(16 * j, 16)]
                        t = t_v[e, pl.ds(16 * j, 16)]
                        n = nr_v[e, pl.ds(_DIM + 16 * j, 16)]
                        d = h - t
                        dj.append(d)
                        nj.append(n)
                        accd = accd + d * n
                    dot = jnp.sum(accd)
                    acca = jnp.zeros((16,), jnp.float32)
                    for j in range(_DIM // 16):
                        r = nr_v[e, pl.ds(16 * j, 16)]
                        s = dj[j] + r - dot * nj[j]
                        acca = acca + jnp.abs(s)
                    res = _GAMMA - jnp.sum(acca)
                    return jnp.where(lane == k, res, acc)

                acc = lax.fori_loop(0, 16, edge, jnp.zeros((16,), jnp.float32))
                o_flat[pl.ds(ci * _C + g * 16, 16)] = acc
                return carry

            lax.fori_loop(0, _C // 16, group, 0)

        issue_idx(0)
        issue_idx(1)
        wait_idx(0)
        issue(0, 0)

        def loop_body(ci, carry):
            @pl.when(ci + 2 < n_chunks)
            def _():
                issue_idx(ci + 2)

            @pl.when(ci + 1 < n_chunks)
            def _():
                wait_idx(ci + 1)
                issue(ci + 1, (ci + 1) % 2)

            wait(ci, ci % 2)
            process(ci, ci % 2)
            return carry

        lax.fori_loop(0, n_chunks, loop_body, 0)

        pltpu.sync_copy(o_flat, out_hbm.at[wid])

    return sc_kernel


def kernel(node_emb, rel_table, norm_table, edge_id, src, dst):
    n_edges = edge_id.shape[0]
    epw = n_edges // _NW
    n_chunks = epw // _C
    relnorm = _prep(rel_table.astype(jnp.float32), norm_table.astype(jnp.float32))
    idx_all = jnp.stack(
        [src.astype(jnp.int32).reshape(_NW, n_chunks, _C),
         dst.astype(jnp.int32).reshape(_NW, n_chunks, _C),
         edge_id.astype(jnp.int32).reshape(_NW, n_chunks, _C)], axis=2)
    out = _make_sc_kernel(n_edges)(node_emb.astype(jnp.float32), relnorm,
                                   idx_all)
    return out.reshape(n_edges)


# speedup vs baseline: 14.3866x; 1.2326x over previous
import functools

import jax
import jax.numpy as jnp
from jax import lax
from jax.experimental import pallas as pl
from jax.experimental.pallas import tpu as pltpu
from jax.experimental.pallas import tpu_sc as plsc

_DIM = 128
_GAMMA = 12.0
_NW = 32
_C = 80


def _prep_body(rel_ref, norm_ref, out_ref):
    x = norm_ref[...]
    ss = jnp.sum(x * x, axis=-1, keepdims=True)
    inv = lax.rsqrt(jnp.maximum(ss, 1e-24))
    out_ref[:, :_DIM] = rel_ref[...]
    out_ref[:, _DIM:] = x * inv


def _prep(rel_table, norm_table):
    n_rel = rel_table.shape[0]
    return pl.pallas_call(
        _prep_body,
        out_shape=jax.ShapeDtypeStruct((n_rel, 2 * _DIM), jnp.float32),
    )(rel_table, norm_table)


@functools.cache
def _make_sc_kernel(n_edges):
    epw = n_edges // _NW
    n_chunks = epw // _C

    mesh = plsc.VectorSubcoreMesh(core_axis_name="c", subcore_axis_name="s")

    @functools.partial(
        pl.kernel,
        out_type=jax.ShapeDtypeStruct((_NW, epw), jnp.float32),
        mesh=mesh,
        compiler_params=pltpu.CompilerParams(needs_layout_passes=False,
                                             use_tc_tiling_on_sc=False),
        scratch_types=[
            pltpu.VMEM((4, 3, _C), jnp.int32),
            pltpu.VMEM((2, _C, _DIM // 2), jnp.int32),
            pltpu.VMEM((2, _C, _DIM // 2), jnp.int32),
            pltpu.VMEM((2, _C, _DIM), jnp.int32),
            pltpu.VMEM((epw,), jnp.float32),
            pltpu.SemaphoreType.DMA((4,)),
            pltpu.SemaphoreType.DMA((2,)),
        ],
    )
    def sc_kernel(node_hbm, relnorm_hbm, idx_hbm, out_hbm,
                  idx4, h2, t2, nr2, o_flat, isems, gsems):
        wid = lax.axis_index("s") * 2 + lax.axis_index("c")

        def issue_idx(ci):
            islot = ci % 4
            pltpu.async_copy(idx_hbm.at[wid, ci], idx4.at[islot],
                             isems.at[islot])

        def wait_idx(ci):
            islot = ci % 4
            pltpu.make_async_copy(idx_hbm.at[wid, ci], idx4.at[islot],
                                  isems.at[islot]).wait()

        def issue(ci, slot):
            islot = ci % 4
            pltpu.async_copy(node_hbm.at[idx4.at[islot, 0]], h2.at[slot],
                             gsems.at[slot])
            pltpu.async_copy(node_hbm.at[idx4.at[islot, 1]], t2.at[slot],
                             gsems.at[slot])
            pltpu.async_copy(relnorm_hbm.at[idx4.at[islot, 2]], nr2.at[slot],
                             gsems.at[slot])

        def wait(ci, slot):
            islot = ci % 4
            pltpu.make_async_copy(node_hbm.at[idx4.at[islot, 0]], h2.at[slot],
                                  gsems.at[slot]).wait()
            pltpu.make_async_copy(node_hbm.at[idx4.at[islot, 1]], t2.at[slot],
                                  gsems.at[slot]).wait()
            pltpu.make_async_copy(relnorm_hbm.at[idx4.at[islot, 2]],
                                  nr2.at[slot], gsems.at[slot]).wait()

        def process(ci, slot):
            h_v, t_v, nr_v = h2.at[slot], t2.at[slot], nr2.at[slot]
            lane = lax.iota(jnp.int32, 16)

            def group(g, carry):
                def edge(k, acc):
                    e = g * 16 + k
                    dj = []
                    nj = []
                    accd = jnp.zeros((32,), jnp.bfloat16)
                    for j in range(_DIM // 32):
                        h = plsc.bitcast(h_v[e, pl.ds(16 * j, 16)],
                                         jnp.bfloat16)
                        t = plsc.bitcast(t_v[e, pl.ds(16 * j, 16)],
                                         jnp.bfloat16)
                        n = plsc.bitcast(
                            nr_v[e, pl.ds(_DIM // 2 + 16 * j, 16)],
                            jnp.bfloat16)
                        d = h - t
                        dj.append(d)
                        nj.append(n)
                        accd = accd + d * n
                    da, db = plsc.unpack(accd,
                                         format=plsc.PackFormat.INTERLEAVED)
                    dot = jnp.sum(da + db)
                    dotv = jnp.broadcast_to(dot, (16,))
                    dotb = plsc.pack(dotv, dotv,
                                     format=plsc.PackFormat.INTERLEAVED)
                    acca = jnp.zeros((32,), jnp.bfloat16)
                    for j in range(_DIM // 32):
                        r = plsc.bitcast(nr_v[e, pl.ds(16 * j, 16)],
                                         jnp.bfloat16)
                        s = dj[j] + r - dotb * nj[j]
                        acca = acca + jnp.abs(s)
                    aa, ab = plsc.unpack(acca,
                                         format=plsc.PackFormat.INTERLEAVED)
                    res = _GAMMA - jnp.sum(aa + ab)
                    return jnp.where(lane == k, res, acc)

                acc = lax.fori_loop(0, 16, edge, jnp.zeros((16,), jnp.float32))
                o_flat[pl.ds(ci * _C + g * 16, 16)] = acc
                return carry

            lax.fori_loop(0, _C // 16, group, 0)

        issue_idx(0)
        issue_idx(1)
        wait_idx(0)
        issue(0, 0)

        def loop_body(ci, carry):
            @pl.when(ci + 2 < n_chunks)
            def _():
                issue_idx(ci + 2)

            @pl.when(ci + 1 < n_chunks)
            def _():
                wait_idx(ci + 1)
                issue(ci + 1, (ci + 1) % 2)

            wait(ci, ci % 2)
            process(ci, ci % 2)
            return carry

        lax.fori_loop(0, n_chunks, loop_body, 0)

        pltpu.sync_copy(o_flat, out_hbm.at[wid])

    return sc_kernel


def _pack_bf16(x):
    n, d = x.shape
    b = x.astype(jnp.bfloat16).reshape(n, d // 2, 2)
    return lax.bitcast_convert_type(b, jnp.int32)


def kernel(node_emb, rel_table, norm_table, edge_id, src, dst):
    n_edges = edge_id.shape[0]
    epw = n_edges // _NW
    n_chunks = epw // _C
    relnorm = _prep(rel_table.astype(jnp.float32), norm_table.astype(jnp.float32))
    idx_all = jnp.stack(
        [src.astype(jnp.int32).reshape(_NW, n_chunks, _C),
         dst.astype(jnp.int32).reshape(_NW, n_chunks, _C),
         edge_id.astype(jnp.int32).reshape(_NW, n_chunks, _C)], axis=2)
    out = _make_sc_kernel(n_edges)(_pack_bf16(node_emb.astype(jnp.float32)),
                                   _pack_bf16(relnorm), idx_all)
    return out.reshape(n_edges)
